# baseline (speedup 1.0000x reference)
import functools

import jax
import jax.numpy as jnp
from jax import lax
from jax.experimental import pallas as pl
from jax.experimental.pallas import tpu as pltpu
from jax.experimental.pallas import tpu_sc as plsc

L = 16


def _build_sc_kernel(batch, height, width, num_pos_feats):
    hw = height * width
    channels = 2 * num_pos_feats
    n_workers = 32
    rows_per_worker = channels // n_workers
    slab_words = rows_per_worker * hw
    col_words = width * num_pos_feats
    row_words = height * num_pos_feats
    mesh = plsc.VectorSubcoreMesh(core_axis_name="c", subcore_axis_name="s")

    @functools.partial(
        pl.kernel,
        mesh=mesh,
        compiler_params=pltpu.CompilerParams(needs_layout_passes=False),
        out_type=jax.ShapeDtypeStruct((batch, channels * hw), jnp.float32),
        scratch_types=[
            pltpu.VMEM((col_words + row_words,), jnp.float32),
            pltpu.VMEM((slab_words,), jnp.float32),
            pltpu.SemaphoreType.DMA,
        ],
    )
    def k(row_flat_hbm, col_flat_hbm, out_hbm, tab_v, slab_v, sem):
        wid = lax.axis_index("s") * 2 + lax.axis_index("c")

        pltpu.sync_copy(col_flat_hbm.at[pl.ds(0, col_words)],
                        tab_v.at[pl.ds(0, col_words)])
        pltpu.sync_copy(row_flat_hbm.at[pl.ds(0, row_words)],
                        tab_v.at[pl.ds(col_words, row_words)])

        wid_v = jnp.full((L,), wid, jnp.int32)
        is_y = wid_v >= (n_workers // 2)
        w_iota = lax.iota(jnp.int32, L)

        x_col = []
        y_col = []
        for j in range(rows_per_worker):
            g = wid_v * rows_per_worker + j
            x_col.append(g)
            y_col.append(g - num_pos_feats)

        def body(h, carry):
            y_base = jnp.full((L,), col_words, jnp.int32) + h * num_pos_feats
            for j in range(rows_per_worker):
                y_flat = y_base + y_col[j]
                for half in range(width // L):
                    x_flat = (w_iota + half * L) * num_pos_feats + x_col[j]
                    idx = jnp.where(is_y, y_flat, x_flat)
                    vals = plsc.load_gather(tab_v, [idx])
                    slab_v[pl.ds(j * hw + h * width + half * L, L)] = vals
            return carry

        lax.fori_loop(0, height, body, 0)

        copies = [
            pltpu.async_copy(
                slab_v,
                out_hbm.at[b, pl.ds(wid * slab_words, slab_words)],
                sem,
            )
            for b in range(batch)
        ]
        for c in copies:
            c.wait()

    return k


def kernel(pixel_values, pixel_mask, row_embed, col_embed):
    batch = pixel_values.shape[0]
    height, width = pixel_values.shape[-2:]
    num_pos_feats = row_embed.shape[-1]
    k = _build_sc_kernel(batch, height, width, num_pos_feats)
    out = k(row_embed.reshape(-1), col_embed.reshape(-1))
    return out.reshape(batch, 2 * num_pos_feats, height, width)


# speedup vs baseline: 1.3280x; 1.3280x over previous
import functools

import jax
import jax.numpy as jnp
from jax import lax
from jax.experimental import pallas as pl
from jax.experimental.pallas import tpu as pltpu
from jax.experimental.pallas import tpu_sc as plsc

L = 16


def _build_sc_kernel(batch, height, width, num_pos_feats):
    hw = height * width
    channels = 2 * num_pos_feats
    n_workers = 32
    rows_per_worker = channels // n_workers
    slab_words = rows_per_worker * hw
    col_words = width * num_pos_feats
    row_words = height * num_pos_feats
    mesh = plsc.VectorSubcoreMesh(core_axis_name="c", subcore_axis_name="s")

    @functools.partial(
        pl.kernel,
        mesh=mesh,
        compiler_params=pltpu.CompilerParams(needs_layout_passes=False),
        out_type=jax.ShapeDtypeStruct((batch, channels, height, width),
                                      jnp.float32),
        scratch_types=[
            pltpu.VMEM((col_words + row_words,), jnp.float32),
            pltpu.VMEM((rows_per_worker, height, width), jnp.float32),
            pltpu.SemaphoreType.DMA,
        ],
    )
    def k(row_flat_hbm, col_flat_hbm, out_hbm, tab_v, slab_v, sem):
        wid = lax.axis_index("s") * 2 + lax.axis_index("c")

        pltpu.sync_copy(col_flat_hbm.at[pl.ds(0, col_words)],
                        tab_v.at[pl.ds(0, col_words)])
        pltpu.sync_copy(row_flat_hbm.at[pl.ds(0, row_words)],
                        tab_v.at[pl.ds(col_words, row_words)])

        wid_v = jnp.full((L,), wid, jnp.int32)
        is_y = wid_v >= (n_workers // 2)
        w_iota = lax.iota(jnp.int32, L)

        x_col = []
        y_col = []
        for j in range(rows_per_worker):
            g = wid_v * rows_per_worker + j
            x_col.append(g)
            y_col.append(g - num_pos_feats)

        def body(h, carry):
            y_base = jnp.full((L,), col_words, jnp.int32) + h * num_pos_feats
            for j in range(rows_per_worker):
                y_flat = y_base + y_col[j]
                for half in range(width // L):
                    x_flat = (w_iota + half * L) * num_pos_feats + x_col[j]
                    idx = jnp.where(is_y, y_flat, x_flat)
                    vals = plsc.load_gather(tab_v, [idx])
                    slab_v[j, h, pl.ds(half * L, L)] = vals
            return carry

        lax.fori_loop(0, height, body, 0)

        copies = [
            pltpu.async_copy(
                slab_v,
                out_hbm.at[b, pl.ds(wid * rows_per_worker, rows_per_worker)],
                sem,
            )
            for b in range(batch)
        ]
        for c in copies:
            c.wait()

    return k


def kernel(pixel_values, pixel_mask, row_embed, col_embed):
    batch = pixel_values.shape[0]
    height, width = pixel_values.shape[-2:]
    num_pos_feats = row_embed.shape[-1]
    k = _build_sc_kernel(batch, height, width, num_pos_feats)
    return k(row_embed.reshape(-1), col_embed.reshape(-1))


# speedup vs baseline: 3.1744x; 2.3904x over previous
import functools

import jax
import jax.numpy as jnp
from jax import lax
from jax.experimental import pallas as pl
from jax.experimental.pallas import tpu as pltpu
from jax.experimental.pallas import tpu_sc as plsc

L = 16


def _build_sc_kernel(batch, height, width, num_pos_feats):
    channels = 2 * num_pos_feats
    n_workers = 32
    h_groups = n_workers // batch
    h_per_group = height // h_groups
    nbuf = 4
    col_words = width * num_pos_feats
    row_words = height * num_pos_feats
    cblocks = channels // L
    xblocks = num_pos_feats // L
    mesh = plsc.VectorSubcoreMesh(core_axis_name="c", subcore_axis_name="s")

    @functools.partial(
        pl.kernel,
        mesh=mesh,
        compiler_params=pltpu.CompilerParams(needs_layout_passes=False),
        out_type=jax.ShapeDtypeStruct((batch, height, width, channels),
                                      jnp.float32),
        scratch_types=[
            pltpu.VMEM((col_words + row_words,), jnp.float32),
            pltpu.VMEM((nbuf, width, channels), jnp.float32),
        ] + [pltpu.SemaphoreType.DMA] * nbuf,
    )
    def k(row_flat_hbm, col_flat_hbm, out_hbm, tab_v, slab_v, *sems):
        wid = lax.axis_index("s") * 2 + lax.axis_index("c")
        b = wid // h_groups
        q = wid % h_groups

        pltpu.sync_copy(col_flat_hbm.at[pl.ds(0, col_words)],
                        tab_v.at[pl.ds(0, col_words)])
        pltpu.sync_copy(row_flat_hbm.at[pl.ds(0, row_words)],
                        tab_v.at[pl.ds(col_words, row_words)])

        copies = [None] * h_per_group
        for hh in range(h_per_group):
            h = q * h_per_group + hh
            slot = hh % nbuf
            if hh >= nbuf:
                copies[hh - nbuf].wait()

            y_base = col_words + h * num_pos_feats
            yv = [tab_v[pl.ds(y_base + cb * L, L)] for cb in range(xblocks)]

            def wbody(w, carry, slot=slot, yv=yv):
                for cb in range(xblocks):
                    slab_v[slot, w, pl.ds(cb * L, L)] = (
                        tab_v[pl.ds(w * num_pos_feats + cb * L, L)])
                for cb in range(xblocks):
                    slab_v[slot, w, pl.ds(num_pos_feats + cb * L, L)] = yv[cb]
                return carry

            lax.fori_loop(0, width, wbody, 0)
            copies[hh] = pltpu.async_copy(
                slab_v.at[slot], out_hbm.at[b, h], sems[slot])

        for hh in range(h_per_group - nbuf, h_per_group):
            copies[hh].wait()

    return k


def kernel(pixel_values, pixel_mask, row_embed, col_embed):
    batch = pixel_values.shape[0]
    height, width = pixel_values.shape[-2:]
    num_pos_feats = row_embed.shape[-1]
    k = _build_sc_kernel(batch, height, width, num_pos_feats)
    out = k(row_embed.reshape(-1), col_embed.reshape(-1))
    return jnp.transpose(out, (0, 3, 1, 2))


# speedup vs baseline: 8.7924x; 2.7698x over previous
import jax
import jax.numpy as jnp
from jax.experimental import pallas as pl
from jax.experimental.pallas import tpu as pltpu


def _body(row_ref, col_ref, out_ref, *, height, width, num_pos_feats, hblk):
    q = pl.program_id(1)
    col = col_ref[:width, :]
    rows = row_ref[pl.ds(q * hblk, hblk), :]
    out_ref[0, :, :, :num_pos_feats] = jnp.broadcast_to(
        col[None, :, :], (hblk, width, num_pos_feats))
    out_ref[0, :, :, num_pos_feats:] = jnp.broadcast_to(
        rows[:, None, :], (hblk, width, num_pos_feats))


def kernel(pixel_values, pixel_mask, row_embed, col_embed):
    batch = pixel_values.shape[0]
    height, width = pixel_values.shape[-2:]
    num_rows, num_pos_feats = row_embed.shape
    channels = 2 * num_pos_feats
    hblk = 8
    grid = (batch, height // hblk)

    import functools
    body = functools.partial(
        _body, height=height, width=width, num_pos_feats=num_pos_feats,
        hblk=hblk)

    out = pl.pallas_call(
        body,
        grid=grid,
        in_specs=[
            pl.BlockSpec((num_rows, num_pos_feats), lambda b, q: (0, 0)),
            pl.BlockSpec((num_rows, num_pos_feats), lambda b, q: (0, 0)),
        ],
        out_specs=pl.BlockSpec((1, hblk, width, channels),
                               lambda b, q: (b, q, 0, 0)),
        out_shape=jax.ShapeDtypeStruct((batch, height, width, channels),
                                       jnp.float32),
        compiler_params=pltpu.CompilerParams(
            dimension_semantics=("parallel", "parallel")),
    )(row_embed, col_embed)
    return jnp.transpose(out, (0, 3, 1, 2))


# speedup vs baseline: 18.7334x; 2.1306x over previous
import jax
import jax.numpy as jnp
from jax.experimental import pallas as pl
from jax.experimental.pallas import tpu as pltpu


def _body(row_ref, col_ref, out_ref, *, height, width, num_pos_feats, nb):
    col = col_ref[:width, :]
    rows = row_ref[:height, :]
    out_ref[:, :, :, :num_pos_feats] = jnp.broadcast_to(
        col[None, None, :, :], (nb, height, width, num_pos_feats))
    out_ref[:, :, :, num_pos_feats:] = jnp.broadcast_to(
        rows[None, :, None, :], (nb, height, width, num_pos_feats))


def kernel(pixel_values, pixel_mask, row_embed, col_embed):
    batch = pixel_values.shape[0]
    height, width = pixel_values.shape[-2:]
    num_rows, num_pos_feats = row_embed.shape
    channels = 2 * num_pos_feats
    nb = 2
    grid = (batch // nb,)

    import functools
    body = functools.partial(
        _body, height=height, width=width, num_pos_feats=num_pos_feats,
        nb=nb)

    out = pl.pallas_call(
        body,
        grid=grid,
        in_specs=[
            pl.BlockSpec((num_rows, num_pos_feats), lambda i: (0, 0)),
            pl.BlockSpec((num_rows, num_pos_feats), lambda i: (0, 0)),
        ],
        out_specs=pl.BlockSpec((nb, height, width, channels),
                               lambda i: (i, 0, 0, 0)),
        out_shape=jax.ShapeDtypeStruct((batch, height, width, channels),
                                       jnp.float32),
        compiler_params=pltpu.CompilerParams(
            dimension_semantics=("parallel",)),
    )(row_embed, col_embed)
    return jnp.transpose(out, (0, 3, 1, 2))
